# Initial kernel scaffold; baseline (speedup 1.0000x reference)
import jax
import jax.numpy as jnp
from jax.experimental import pallas as pl


def kernel(x, edge_index, edge_attr, W1, att_src1, att_dst1, b1, W2, att_src2, att_dst2, b2):
    raise NotImplementedError("write your pallas kernel here")


# speedup vs baseline: 24.5052x; 24.5052x over previous
import jax
import jax.numpy as jnp
from jax import lax
from jax.experimental import pallas as pl
from jax.experimental.pallas import tpu as pltpu
from jax.experimental.pallas import tpu_sc as plsc

_N = 10000
_E = 320000
_DIN = 128
_H1 = 8
_C1 = 8
_HC1 = 64
_DOUT = 128

_NP = 10240
_NC = 2
_NS = 16
_NW = _NC * _NS
_CH = 128
_NCH = 81
_EPAD = _NW * _NCH * _CH
_RS = _NP // _NS

_f32 = jnp.float32
_i32 = jnp.int32

_mesh = plsc.VectorSubcoreMesh(core_axis_name="c", subcore_axis_name="s",
                               num_cores=_NC, num_subcores=_NS)
_sc_params = pltpu.CompilerParams(use_tc_tiling_on_sc=False)


def _sc_logits_body(ts_hbm, td_hbm, si_hbm, di_hbm, p_hbm, dp_hbm,
                    sidx, didx, sv, dv, pv, den_sh, sem0, sem1):
  cid = lax.axis_index("c")
  sid = lax.axis_index("s")
  wid = sid * _NC + cid

  def _z(k, c):
    pv[k, :] = jnp.zeros((16,), _f32)
    return c
  lax.fori_loop(0, _CH, _z, 0, unroll=8)
  for i in range(_RS // _CH):
    pltpu.sync_copy(pv, den_sh.at[pl.ds(sid * _RS + i * _CH, _CH)])
  plsc.subcore_barrier()

  base = wid * _NCH * _CH

  def _chunk(j, c):
    off = base + j * _CH
    pltpu.sync_copy(si_hbm.at[pl.ds(off, _CH)], sidx)
    pltpu.sync_copy(di_hbm.at[pl.ds(off, _CH)], didx)
    cp1 = pltpu.async_copy(ts_hbm.at[sidx], sv, sem0)
    cp2 = pltpu.async_copy(td_hbm.at[didx], dv, sem1)
    cp1.wait()
    cp2.wait()

    def _vec(k, c2):
      e = sv[k, :] + dv[k, :]
      e = jnp.where(e >= 0.0, e, 0.2 * e)
      pv[k, :] = jnp.exp(e)
      return c2
    lax.fori_loop(0, _CH, _vec, 0, unroll=8)

    pltpu.sync_copy(pv, p_hbm.at[pl.ds(off, _CH)])
    pltpu.sync_copy(pv, den_sh.at[didx], add=True)
    return c
  lax.fori_loop(0, _NCH, _chunk, 0)

  plsc.subcore_barrier()
  pltpu.sync_copy(den_sh.at[pl.ds(sid * _RS, _RS)],
                  dp_hbm.at[cid, pl.ds(sid * _RS, _RS)])


_sc_logits = pl.kernel(
    _sc_logits_body,
    out_type=(jax.ShapeDtypeStruct((_EPAD, 16), _f32),
              jax.ShapeDtypeStruct((_NC, _NP, 16), _f32)),
    mesh=_mesh,
    compiler_params=_sc_params,
    scratch_types=[
        pltpu.VMEM((_CH,), _i32),
        pltpu.VMEM((_CH,), _i32),
        pltpu.VMEM((_CH, 16), _f32),
        pltpu.VMEM((_CH, 16), _f32),
        pltpu.VMEM((_CH, 16), _f32),
        pltpu.VMEM_SHARED((_NP, 16), _f32),
        pltpu.SemaphoreType.DMA,
        pltpu.SemaphoreType.DMA,
    ],
)


def _make_sc_msg(D, EPR, PCOLS):
  nt = D // 16
  prows = _CH // EPR
  ew = PCOLS // EPR

  def body(h_hbm, pe_hbm, si_hbm, di_hbm, op_hbm,
           sidx, didx, rows, pv, out_sh, sem0):
    cid = lax.axis_index("c")
    sid = lax.axis_index("s")
    wid = sid * _NC + cid

    def _z(k, c):
      for t in range(nt):
        rows[k, pl.ds(16 * t, 16)] = jnp.zeros((16,), _f32)
      return c
    lax.fori_loop(0, _CH, _z, 0, unroll=4)
    for i in range(_RS // _CH):
      pltpu.sync_copy(rows, out_sh.at[pl.ds(sid * _RS + i * _CH, _CH)])
    plsc.subcore_barrier()

    base = wid * _NCH * _CH

    def _chunk(j, c):
      off = base + j * _CH
      pltpu.sync_copy(si_hbm.at[pl.ds(off, _CH)], sidx)
      pltpu.sync_copy(di_hbm.at[pl.ds(off, _CH)], didx)
      cp = pltpu.async_copy(h_hbm.at[sidx], rows, sem0)
      pltpu.sync_copy(pe_hbm.at[pl.ds(off // EPR, prows)], pv)
      cp.wait()

      def _vec(q, c2):
        for m in range(EPR):
          k = q * EPR + m
          for t in range(nt):
            a = pv[q, pl.ds(m * ew + 16 * ((16 * t) % ew // 16), 16)]
            rows[k, pl.ds(16 * t, 16)] = rows[k, pl.ds(16 * t, 16)] * a
        return c2
      lax.fori_loop(0, prows, _vec, 0, unroll=2)

      pltpu.sync_copy(rows, out_sh.at[didx], add=True)
      return c
    lax.fori_loop(0, _NCH, _chunk, 0)

    plsc.subcore_barrier()
    pltpu.sync_copy(out_sh.at[pl.ds(sid * _RS, _RS)],
                    op_hbm.at[cid, pl.ds(sid * _RS, _RS)])

  return pl.kernel(
      body,
      out_type=jax.ShapeDtypeStruct((_NC, _NP, D), _f32),
      mesh=_mesh,
      compiler_params=_sc_params,
      scratch_types=[
          pltpu.VMEM((_CH,), _i32),
          pltpu.VMEM((_CH,), _i32),
          pltpu.VMEM((_CH, D), _f32),
          pltpu.VMEM((_CH // EPR, PCOLS), _f32),
          pltpu.VMEM_SHARED((_NP, D), _f32),
          pltpu.SemaphoreType.DMA,
      ],
  )


_sc_msg1 = _make_sc_msg(_HC1, 2, 128)
_sc_msg2 = _make_sc_msg(_DOUT, 1, 16)


_BN = 256
_NB = _NP // _BN


def _tc1_body(x_ref, w1_ref, aa_ref, ab_ref, h_ref, ts_ref, td_ref):
  h = jnp.dot(x_ref[...], w1_ref[...], preferred_element_type=_f32)
  h_ref[...] = h
  ts_ref[...] = jnp.dot(h, aa_ref[...], preferred_element_type=_f32)
  td_ref[...] = jnp.dot(h, ab_ref[...], preferred_element_type=_f32)


_tc1 = pl.pallas_call(
    _tc1_body,
    grid=(_NB,),
    in_specs=[
        pl.BlockSpec((_BN, _DIN), lambda i: (i, 0)),
        pl.BlockSpec((_DIN, _HC1), lambda i: (0, 0)),
        pl.BlockSpec((_HC1, 16), lambda i: (0, 0)),
        pl.BlockSpec((_HC1, 16), lambda i: (0, 0)),
    ],
    out_specs=[
        pl.BlockSpec((_BN, _HC1), lambda i: (i, 0)),
        pl.BlockSpec((_BN, 16), lambda i: (i, 0)),
        pl.BlockSpec((_BN, 16), lambda i: (i, 0)),
    ],
    out_shape=[
        jax.ShapeDtypeStruct((_NP, _HC1), _f32),
        jax.ShapeDtypeStruct((_NP, 16), _f32),
        jax.ShapeDtypeStruct((_NP, 16), _f32),
    ],
)


def _tce_body(p_ref, r_ref, o_ref):
  o_ref[...] = jnp.dot(p_ref[...], r_ref[...], preferred_element_type=_f32)


_EB = 512
_tce = pl.pallas_call(
    _tce_body,
    grid=(_EPAD // 2 // _EB,),
    in_specs=[
        pl.BlockSpec((_EB, 32), lambda i: (i, 0)),
        pl.BlockSpec((32, 128), lambda i: (0, 0)),
    ],
    out_specs=pl.BlockSpec((_EB, 128), lambda i: (i, 0)),
    out_shape=jax.ShapeDtypeStruct((_EPAD // 2, 128), _f32),
)


def _tc2_body(o0_ref, o1_ref, d0_ref, d1_ref, r8_ref, b1_ref, w2_ref,
              as_ref, ad_ref, h2_ref, t2s_ref, t2d_ref):
  o = o0_ref[...] + o1_ref[...]
  d = d0_ref[...] + d1_ref[...]
  dex = jnp.dot(d, r8_ref[...], preferred_element_type=_f32)
  out1 = o / (dex + 1e-16) + b1_ref[...]
  hh = jnp.where(out1 > 0.0, out1, jnp.exp(out1) - 1.0)
  h2 = jnp.dot(hh, w2_ref[...], preferred_element_type=_f32)
  h2_ref[...] = h2
  t2s_ref[...] = jnp.dot(h2, as_ref[...], preferred_element_type=_f32)
  t2d_ref[...] = jnp.dot(h2, ad_ref[...], preferred_element_type=_f32)


_tc2 = pl.pallas_call(
    _tc2_body,
    grid=(_NB,),
    in_specs=[
        pl.BlockSpec((_BN, _HC1), lambda i: (i, 0)),
        pl.BlockSpec((_BN, _HC1), lambda i: (i, 0)),
        pl.BlockSpec((_BN, 16), lambda i: (i, 0)),
        pl.BlockSpec((_BN, 16), lambda i: (i, 0)),
        pl.BlockSpec((16, _HC1), lambda i: (0, 0)),
        pl.BlockSpec((1, _HC1), lambda i: (0, 0)),
        pl.BlockSpec((_HC1, _DOUT), lambda i: (0, 0)),
        pl.BlockSpec((_DOUT, 16), lambda i: (0, 0)),
        pl.BlockSpec((_DOUT, 16), lambda i: (0, 0)),
    ],
    out_specs=[
        pl.BlockSpec((_BN, _DOUT), lambda i: (i, 0)),
        pl.BlockSpec((_BN, 16), lambda i: (i, 0)),
        pl.BlockSpec((_BN, 16), lambda i: (i, 0)),
    ],
    out_shape=[
        jax.ShapeDtypeStruct((_NP, _DOUT), _f32),
        jax.ShapeDtypeStruct((_NP, 16), _f32),
        jax.ShapeDtypeStruct((_NP, 16), _f32),
    ],
)


def _tc3_body(o0_ref, o1_ref, d0_ref, d1_ref, s_ref, b2_ref, out_ref):
  o = o0_ref[...] + o1_ref[...]
  d = d0_ref[...] + d1_ref[...]
  dex = jnp.dot(d, s_ref[...], preferred_element_type=_f32)
  out_ref[...] = o / (dex + 1e-16) + b2_ref[...]


_tc3 = pl.pallas_call(
    _tc3_body,
    grid=(_NB,),
    in_specs=[
        pl.BlockSpec((_BN, _DOUT), lambda i: (i, 0)),
        pl.BlockSpec((_BN, _DOUT), lambda i: (i, 0)),
        pl.BlockSpec((_BN, 16), lambda i: (i, 0)),
        pl.BlockSpec((_BN, 16), lambda i: (i, 0)),
        pl.BlockSpec((16, _DOUT), lambda i: (0, 0)),
        pl.BlockSpec((1, _DOUT), lambda i: (0, 0)),
    ],
    out_specs=pl.BlockSpec((_BN, _DOUT), lambda i: (i, 0)),
    out_shape=jax.ShapeDtypeStruct((_NP, _DOUT), _f32),
)


def kernel(x, edge_index, edge_attr, W1, att_src1, att_dst1, b1,
           W2, att_src2, att_dst2, b2):
  del edge_attr

  loop = jnp.arange(_N, dtype=_i32)
  npad = _EPAD - (_E + _N)
  pad = jnp.full((npad,), _N, _i32)
  src = jnp.concatenate([edge_index[0].astype(_i32), loop, pad])
  dst = jnp.concatenate([edge_index[1].astype(_i32), loop, pad])
  xp = jnp.zeros((_NP, _DIN), _f32).at[:_N].set(x)

  m = jnp.repeat(jnp.eye(_H1, dtype=_f32), _C1, axis=0)
  a1s = jnp.pad(m * att_src1.reshape(-1)[:, None], ((0, 0), (0, 8)))
  a1d = jnp.pad(m * att_dst1.reshape(-1)[:, None], ((0, 0), (0, 8)))
  rexp = jnp.pad(jnp.kron(jnp.eye(_H1, dtype=_f32), jnp.ones((1, _C1), _f32)),
                 ((0, 8), (0, 0)))
  rexp2 = jnp.kron(jnp.eye(2, dtype=_f32), rexp)
  r8 = rexp
  a2s = att_src2.reshape(-1)[:, None] * jnp.ones((1, 16), _f32)
  a2d = att_dst2.reshape(-1)[:, None] * jnp.ones((1, 16), _f32)
  s16 = jnp.zeros((16, _DOUT), _f32).at[0].set(1.0)
  b1r = b1.reshape(1, _HC1)
  b2r = b2.reshape(1, _DOUT)

  h1, t1s, t1d = _tc1(xp, W1, a1s, a1d)
  p1, d1p = _sc_logits(t1s, t1d, src, dst)
  pe1 = _tce(p1.reshape(_EPAD // 2, 32), rexp2)
  o1p = _sc_msg1(h1, pe1, src, dst)
  h2, t2s, t2d = _tc2(o1p[0], o1p[1], d1p[0], d1p[1], r8, b1r, W2, a2s, a2d)

  p2, d2p = _sc_logits(t2s, t2d, src, dst)
  o2p = _sc_msg2(h2, p2, src, dst)
  out = _tc3(o2p[0], o2p[1], d2p[0], d2p[1], s16, b2r)
  return out[:_N]


# speedup vs baseline: 42.5068x; 1.7346x over previous
import jax
import jax.numpy as jnp
from jax import lax
from jax.experimental import pallas as pl
from jax.experimental.pallas import tpu as pltpu
from jax.experimental.pallas import tpu_sc as plsc

_N = 10000
_E = 320000
_DIN = 128
_H1 = 8
_C1 = 8
_HC1 = 64
_DOUT = 128

_NP = 10240
_NC = 2
_NS = 16
_NW = _NC * _NS
_CH = 128
_NCH = 81
_EPAD = _NW * _NCH * _CH
_RS = _NP // _NS

_f32 = jnp.float32
_i32 = jnp.int32

_mesh = plsc.VectorSubcoreMesh(core_axis_name="c", subcore_axis_name="s",
                               num_cores=_NC, num_subcores=_NS)
_sc_params = pltpu.CompilerParams(use_tc_tiling_on_sc=False)


def _run_pipeline(nch, li, wi, lg, wg, cp, ls, ws):
  assert nch >= 6 and nch % 3 == 0

  def step(jj, a, a1, a2):
    wi(jj + 1, a1)
    lg(jj + 1, a1)
    ws(jj - 1, a2)
    li(jj + 2, a2)
    wg(jj, a)
    cp(a)
    ls(jj, a)

  li(0, 0)
  li(1, 1)
  li(2, 2)
  wi(0, 0)
  lg(0, 0)
  wi(1, 1)
  lg(1, 1)
  wg(0, 0)
  cp(0)
  ls(0, 0)
  wi(2, 2)
  lg(2, 2)
  ws(0, 0)
  li(3, 0)
  wg(1, 1)
  cp(1)
  ls(1, 1)
  wi(3, 0)
  lg(3, 0)
  ws(1, 1)
  li(4, 1)
  wg(2, 2)
  cp(2)
  ls(2, 2)

  def _body(m, c):
    j = 3 + 3 * m
    step(j, 0, 1, 2)
    step(j + 1, 1, 2, 0)
    step(j + 2, 2, 0, 1)
    return c
  lax.fori_loop(0, (nch - 6) // 3, _body, 0)

  step(nch - 3, 0, 1, 2)
  wi(nch - 1, 2)
  lg(nch - 1, 2)
  ws(nch - 3, 0)
  wg(nch - 2, 1)
  cp(1)
  ls(nch - 2, 1)
  ws(nch - 2, 1)
  wg(nch - 1, 2)
  cp(2)
  ls(nch - 1, 2)
  ws(nch - 1, 2)


def _sc_logits_body(ts_hbm, td_hbm, si_hbm, di_hbm, p_hbm, dp_hbm,
                    si0, si1, si2, di0, di1, di2,
                    sv0, sv1, sv2, dv0, dv1, dv2, pv0, pv1, pv2,
                    den_sh, *sems):
  cid = lax.axis_index("c")
  sid = lax.axis_index("s")
  wid = sid * _NC + cid
  sidxb, didxb = (si0, si1, si2), (di0, di1, di2)
  svb, dvb, pvb = (sv0, sv1, sv2), (dv0, dv1, dv2), (pv0, pv1, pv2)
  sib, sgb = sems[0:3], sems[3:6]
  ssb, stb = sems[6:9], sems[9:12]

  def _z(k, c):
    pv0[k, :] = jnp.zeros((16,), _f32)
    return c
  lax.fori_loop(0, _CH, _z, 0, unroll=8)
  for i in range(_RS // _CH):
    pltpu.sync_copy(pv0, den_sh.at[pl.ds(sid * _RS + i * _CH, _CH)])
  plsc.subcore_barrier()

  base = wid * _NCH

  def li(j, s):
    pltpu.async_copy(si_hbm.at[pl.ds((base + j) * _CH, _CH)], sidxb[s],
                     sib[s])
    pltpu.async_copy(di_hbm.at[pl.ds((base + j) * _CH, _CH)], didxb[s],
                     sib[s])

  def wi(j, s):
    pltpu.make_async_copy(si_hbm.at[pl.ds((base + j) * _CH, _CH)], sidxb[s],
                          sib[s]).wait()
    pltpu.make_async_copy(di_hbm.at[pl.ds((base + j) * _CH, _CH)], didxb[s],
                          sib[s]).wait()

  def lg(j, s):
    pltpu.async_copy(ts_hbm.at[sidxb[s]], svb[s], sgb[s])
    pltpu.async_copy(td_hbm.at[didxb[s]], dvb[s], sgb[s])

  def wg(j, s):
    pltpu.make_async_copy(ts_hbm.at[sidxb[s]], svb[s], sgb[s]).wait()
    pltpu.make_async_copy(td_hbm.at[didxb[s]], dvb[s], sgb[s]).wait()

  def cp(s):
    sv_, dv_, pv_ = svb[s], dvb[s], pvb[s]
    def _vec(k, c2):
      e = sv_[k, :] + dv_[k, :]
      pv_[k, :] = jnp.exp(jnp.maximum(e, 0.2 * e))
      return c2
    lax.fori_loop(0, _CH, _vec, 0, unroll=8)

  def ls(j, s):
    pltpu.async_copy(pvb[s], p_hbm.at[pl.ds((base + j) * _CH, _CH)], stb[s])
    pltpu.async_copy(pvb[s], den_sh.at[didxb[s]], ssb[s], add=True)

  def ws(j, s):
    pltpu.make_async_copy(pvb[s], p_hbm.at[pl.ds((base + j) * _CH, _CH)],
                          stb[s]).wait()
    pltpu.make_async_copy(pvb[s], den_sh.at[didxb[s]], ssb[s]).wait()

  _run_pipeline(_NCH, li, wi, lg, wg, cp, ls, ws)

  plsc.subcore_barrier()
  pltpu.sync_copy(den_sh.at[pl.ds(sid * _RS, _RS)],
                  dp_hbm.at[cid, pl.ds(sid * _RS, _RS)])


_sc_logits = pl.kernel(
    _sc_logits_body,
    out_type=(jax.ShapeDtypeStruct((_EPAD, 16), _f32),
              jax.ShapeDtypeStruct((_NC, _NP, 16), _f32)),
    mesh=_mesh,
    compiler_params=_sc_params,
    scratch_types=(
        [pltpu.VMEM((_CH,), _i32)] * 6
        + [pltpu.VMEM((_CH, 16), _f32)] * 9
        + [pltpu.VMEM_SHARED((_NP, 16), _f32)]
        + [pltpu.SemaphoreType.DMA] * 12
    ),
)


def _make_sc_msg(D, EPR, PCOLS, CH):
  nt = D // 16
  prows = CH // EPR
  ew = PCOLS // EPR
  nch = (_NCH * _CH) // CH
  zr = _RS // CH
  assert _RS % CH == 0 and (_NCH * _CH) % CH == 0

  def body(h_hbm, pe_hbm, si_hbm, di_hbm, op_hbm,
           si0, si1, si2, di0, di1, di2,
           rows0, rows1, rows2, pv0, pv1, pv2, out_sh, *sems):
    cid = lax.axis_index("c")
    sid = lax.axis_index("s")
    wid = sid * _NC + cid
    sidxb, didxb = (si0, si1, si2), (di0, di1, di2)
    rowsb, pvb = (rows0, rows1, rows2), (pv0, pv1, pv2)
    sib, sgb = sems[0:3], sems[3:6]
    ssb, stb = sems[6:9], sems[9:12]

    def _z(k, c):
      for t in range(nt):
        rows0[k, pl.ds(16 * t, 16)] = jnp.zeros((16,), _f32)
      return c
    lax.fori_loop(0, CH, _z, 0, unroll=4)
    for i in range(zr):
      pltpu.sync_copy(rows0, out_sh.at[pl.ds(sid * _RS + i * CH, CH)])
    plsc.subcore_barrier()

    base = wid * nch

    def li(j, s):
      pltpu.async_copy(si_hbm.at[pl.ds((base + j) * CH, CH)], sidxb[s],
                       sib[s])
      pltpu.async_copy(di_hbm.at[pl.ds((base + j) * CH, CH)], didxb[s],
                       sib[s])

    def wi(j, s):
      pltpu.make_async_copy(si_hbm.at[pl.ds((base + j) * CH, CH)], sidxb[s],
                            sib[s]).wait()
      pltpu.make_async_copy(di_hbm.at[pl.ds((base + j) * CH, CH)], didxb[s],
                            sib[s]).wait()

    def lg(j, s):
      pltpu.async_copy(h_hbm.at[sidxb[s]], rowsb[s], sgb[s])
      pltpu.async_copy(pe_hbm.at[pl.ds((base + j) * prows, prows)],
                       pvb[s], stb[s])

    def wg(j, s):
      pltpu.make_async_copy(h_hbm.at[sidxb[s]], rowsb[s], sgb[s]).wait()
      pltpu.make_async_copy(pe_hbm.at[pl.ds((base + j) * prows, prows)],
                            pvb[s], stb[s]).wait()

    def cp(s):
      rows_, pv_ = rowsb[s], pvb[s]
      def _vec(q, c2):
        for m in range(EPR):
          k = q * EPR + m
          if ew >= 16 * nt:
            for t in range(nt):
              a = pv_[q, pl.ds(m * ew + 16 * t, 16)]
              rows_[k, pl.ds(16 * t, 16)] = rows_[k, pl.ds(16 * t, 16)] * a
          else:
            a = pv_[q, pl.ds(m * ew, 16)]
            for t in range(nt):
              rows_[k, pl.ds(16 * t, 16)] = rows_[k, pl.ds(16 * t, 16)] * a
        return c2
      lax.fori_loop(0, prows, _vec, 0, unroll=2)

    def ls(j, s):
      pltpu.async_copy(rowsb[s], out_sh.at[didxb[s]], ssb[s], add=True)

    def ws(j, s):
      pltpu.make_async_copy(rowsb[s], out_sh.at[didxb[s]], ssb[s]).wait()

    _run_pipeline(nch, li, wi, lg, wg, cp, ls, ws)

    plsc.subcore_barrier()
    pltpu.sync_copy(out_sh.at[pl.ds(sid * _RS, _RS)],
                    op_hbm.at[cid, pl.ds(sid * _RS, _RS)])

  return pl.kernel(
      body,
      out_type=jax.ShapeDtypeStruct((_NC, _NP, D), _f32),
      mesh=_mesh,
      compiler_params=_sc_params,
      scratch_types=(
          [pltpu.VMEM((CH,), _i32)] * 6
          + [pltpu.VMEM((CH, D), _f32)] * 3
          + [pltpu.VMEM((prows, PCOLS), _f32)] * 3
          + [pltpu.VMEM_SHARED((_NP, D), _f32)]
          + [pltpu.SemaphoreType.DMA] * 12
      ),
  )


_sc_msg1 = _make_sc_msg(_HC1, 2, 128, 128)
_sc_msg2 = _make_sc_msg(_DOUT, 1, 16, 64)


_BN = 256
_NB = _NP // _BN


def _tc1_body(x_ref, w1_ref, aa_ref, ab_ref, h_ref, ts_ref, td_ref):
  h = jnp.dot(x_ref[...], w1_ref[...], preferred_element_type=_f32)
  h_ref[...] = h
  ts_ref[...] = jnp.dot(h, aa_ref[...], preferred_element_type=_f32)
  td_ref[...] = jnp.dot(h, ab_ref[...], preferred_element_type=_f32)


_tc1 = pl.pallas_call(
    _tc1_body,
    grid=(_NB,),
    in_specs=[
        pl.BlockSpec((_BN, _DIN), lambda i: (i, 0)),
        pl.BlockSpec((_DIN, _HC1), lambda i: (0, 0)),
        pl.BlockSpec((_HC1, 16), lambda i: (0, 0)),
        pl.BlockSpec((_HC1, 16), lambda i: (0, 0)),
    ],
    out_specs=[
        pl.BlockSpec((_BN, _HC1), lambda i: (i, 0)),
        pl.BlockSpec((_BN, 16), lambda i: (i, 0)),
        pl.BlockSpec((_BN, 16), lambda i: (i, 0)),
    ],
    out_shape=[
        jax.ShapeDtypeStruct((_NP, _HC1), _f32),
        jax.ShapeDtypeStruct((_NP, 16), _f32),
        jax.ShapeDtypeStruct((_NP, 16), _f32),
    ],
)


def _tce_body(p_ref, r_ref, o_ref):
  o_ref[...] = jnp.dot(p_ref[...], r_ref[...], preferred_element_type=_f32)


_EB = 512
_tce = pl.pallas_call(
    _tce_body,
    grid=(_EPAD // 2 // _EB,),
    in_specs=[
        pl.BlockSpec((_EB, 32), lambda i: (i, 0)),
        pl.BlockSpec((32, 128), lambda i: (0, 0)),
    ],
    out_specs=pl.BlockSpec((_EB, 128), lambda i: (i, 0)),
    out_shape=jax.ShapeDtypeStruct((_EPAD // 2, 128), _f32),
)


def _tc2_body(o0_ref, o1_ref, d0_ref, d1_ref, r8_ref, b1_ref, w2_ref,
              as_ref, ad_ref, h2_ref, t2s_ref, t2d_ref):
  o = o0_ref[...] + o1_ref[...]
  d = d0_ref[...] + d1_ref[...]
  dex = jnp.dot(d, r8_ref[...], preferred_element_type=_f32)
  out1 = o / (dex + 1e-16) + b1_ref[...]
  hh = jnp.where(out1 > 0.0, out1, jnp.exp(out1) - 1.0)
  h2 = jnp.dot(hh, w2_ref[...], preferred_element_type=_f32)
  h2_ref[...] = h2
  t2s_ref[...] = jnp.dot(h2, as_ref[...], preferred_element_type=_f32)
  t2d_ref[...] = jnp.dot(h2, ad_ref[...], preferred_element_type=_f32)


_tc2 = pl.pallas_call(
    _tc2_body,
    grid=(_NB,),
    in_specs=[
        pl.BlockSpec((_BN, _HC1), lambda i: (i, 0)),
        pl.BlockSpec((_BN, _HC1), lambda i: (i, 0)),
        pl.BlockSpec((_BN, 16), lambda i: (i, 0)),
        pl.BlockSpec((_BN, 16), lambda i: (i, 0)),
        pl.BlockSpec((16, _HC1), lambda i: (0, 0)),
        pl.BlockSpec((1, _HC1), lambda i: (0, 0)),
        pl.BlockSpec((_HC1, _DOUT), lambda i: (0, 0)),
        pl.BlockSpec((_DOUT, 16), lambda i: (0, 0)),
        pl.BlockSpec((_DOUT, 16), lambda i: (0, 0)),
    ],
    out_specs=[
        pl.BlockSpec((_BN, _DOUT), lambda i: (i, 0)),
        pl.BlockSpec((_BN, 16), lambda i: (i, 0)),
        pl.BlockSpec((_BN, 16), lambda i: (i, 0)),
    ],
    out_shape=[
        jax.ShapeDtypeStruct((_NP, _DOUT), _f32),
        jax.ShapeDtypeStruct((_NP, 16), _f32),
        jax.ShapeDtypeStruct((_NP, 16), _f32),
    ],
)


def _tc3_body(o0_ref, o1_ref, d0_ref, d1_ref, s_ref, b2_ref, out_ref):
  o = o0_ref[...] + o1_ref[...]
  d = d0_ref[...] + d1_ref[...]
  dex = jnp.dot(d, s_ref[...], preferred_element_type=_f32)
  out_ref[...] = o / (dex + 1e-16) + b2_ref[...]


_tc3 = pl.pallas_call(
    _tc3_body,
    grid=(_NB,),
    in_specs=[
        pl.BlockSpec((_BN, _DOUT), lambda i: (i, 0)),
        pl.BlockSpec((_BN, _DOUT), lambda i: (i, 0)),
        pl.BlockSpec((_BN, 16), lambda i: (i, 0)),
        pl.BlockSpec((_BN, 16), lambda i: (i, 0)),
        pl.BlockSpec((16, _DOUT), lambda i: (0, 0)),
        pl.BlockSpec((1, _DOUT), lambda i: (0, 0)),
    ],
    out_specs=pl.BlockSpec((_BN, _DOUT), lambda i: (i, 0)),
    out_shape=jax.ShapeDtypeStruct((_NP, _DOUT), _f32),
)


def kernel(x, edge_index, edge_attr, W1, att_src1, att_dst1, b1,
           W2, att_src2, att_dst2, b2):
  del edge_attr

  loop = jnp.arange(_N, dtype=_i32)
  npad = _EPAD - (_E + _N)
  pad = jnp.full((npad,), _N, _i32)
  src = jnp.concatenate([edge_index[0].astype(_i32), loop, pad])
  dst = jnp.concatenate([edge_index[1].astype(_i32), loop, pad])
  xp = jnp.zeros((_NP, _DIN), _f32).at[:_N].set(x)

  m = jnp.repeat(jnp.eye(_H1, dtype=_f32), _C1, axis=0)
  a1s = jnp.pad(m * att_src1.reshape(-1)[:, None], ((0, 0), (0, 8)))
  a1d = jnp.pad(m * att_dst1.reshape(-1)[:, None], ((0, 0), (0, 8)))
  rexp = jnp.pad(jnp.kron(jnp.eye(_H1, dtype=_f32), jnp.ones((1, _C1), _f32)),
                 ((0, 8), (0, 0)))
  rexp2 = jnp.kron(jnp.eye(2, dtype=_f32), rexp)
  r8 = rexp
  a2s = att_src2.reshape(-1)[:, None] * jnp.ones((1, 16), _f32)
  a2d = att_dst2.reshape(-1)[:, None] * jnp.ones((1, 16), _f32)
  s16 = jnp.zeros((16, _DOUT), _f32).at[0].set(1.0)
  b1r = b1.reshape(1, _HC1)
  b2r = b2.reshape(1, _DOUT)

  h1, t1s, t1d = _tc1(xp, W1, a1s, a1d)
  p1, d1p = _sc_logits(t1s, t1d, src, dst)
  pe1 = _tce(p1.reshape(_EPAD // 2, 32), rexp2)
  o1p = _sc_msg1(h1, pe1, src, dst)
  h2, t2s, t2d = _tc2(o1p[0], o1p[1], d1p[0], d1p[1], r8, b1r, W2, a2s, a2d)

  p2, d2p = _sc_logits(t2s, t2d, src, dst)
  o2p = _sc_msg2(h2, p2, src, dst)
  out = _tc3(o2p[0], o2p[1], d2p[0], d2p[1], s16, b2r)
  return out[:_N]
